# Initial kernel scaffold; baseline (speedup 1.0000x reference)
import jax
import jax.numpy as jnp
from jax.experimental import pallas as pl


def kernel(x, edge_index, W, b):
    raise NotImplementedError("write your pallas kernel here")


# speedup vs baseline: 1.2797x; 1.2797x over previous
import functools

import jax
import jax.numpy as jnp
from jax import lax
from jax.experimental import pallas as pl
from jax.experimental.pallas import tpu as pltpu
from jax.experimental.pallas import tpu_sc as plsc

N = 10000
C = 128
K = 32
NW = 32
CH_NODES = 4
CH_ROWS = CH_NODES * K
NPW = 320
CHUNKS = NPW // CH_NODES
LANES = 16


def _segmax_body(idx_hbm, x_hbm, out_hbm, idx_v, gbuf, out_v, sem):
    nc = 2
    wid = lax.axis_index("s") * nc + lax.axis_index("c")
    pltpu.sync_copy(idx_hbm.at[wid], idx_v)

    def chunk(j, carry):
        pltpu.async_copy(x_hbm.at[idx_v.at[j]], gbuf, sem).wait()
        for nd in range(CH_NODES):
            for cc in range(C // LANES):
                sl = pl.ds(cc * LANES, LANES)
                vals = [gbuf[nd * K + r, sl] for r in range(K)]
                while len(vals) > 1:
                    nxt = [jnp.maximum(vals[2 * i], vals[2 * i + 1])
                           for i in range(len(vals) // 2)]
                    if len(vals) % 2:
                        nxt.append(vals[-1])
                    vals = nxt
                out_v[j * CH_NODES + nd, sl] = vals[0]
        return carry

    lax.fori_loop(0, CHUNKS, chunk, 0)
    pltpu.sync_copy(out_v, out_hbm.at[pl.ds(wid * NPW, NPW)])


_segmax = functools.partial(
    pl.kernel,
    out_type=jax.ShapeDtypeStruct((NW * NPW, C), jnp.float32),
    mesh=plsc.VectorSubcoreMesh(core_axis_name="c", subcore_axis_name="s"),
    scratch_types=[
        pltpu.VMEM((CHUNKS, CH_ROWS), jnp.int32),
        pltpu.VMEM((CH_ROWS, C), jnp.float32),
        pltpu.VMEM((NPW, C), jnp.float32),
        pltpu.SemaphoreType.DMA,
    ],
)(_segmax_body)


def _mm_body(x_ref, m_ref, wa_ref, wb_ref, b_ref, o_ref):
    acc = jnp.dot(x_ref[...], wa_ref[...], preferred_element_type=jnp.float32)
    acc = acc + jnp.dot(m_ref[...], wb_ref[...], preferred_element_type=jnp.float32)
    o_ref[...] = jnp.maximum(acc + b_ref[...], 0.0)


def _matmul(x, m, wa, wb, b2d):
    blk = 1000
    grid = (N // blk,)
    return pl.pallas_call(
        _mm_body,
        grid=grid,
        in_specs=[
            pl.BlockSpec((blk, C), lambda i: (i, 0)),
            pl.BlockSpec((blk, C), lambda i: (i, 0)),
            pl.BlockSpec((C, C), lambda i: (0, 0)),
            pl.BlockSpec((C, C), lambda i: (0, 0)),
            pl.BlockSpec((1, C), lambda i: (0, 0)),
        ],
        out_specs=pl.BlockSpec((blk, C), lambda i: (i, 0)),
        out_shape=jax.ShapeDtypeStruct((N, C), jnp.float32),
    )(x, m, wa, wb, b2d)


def kernel(x, edge_index, W, b):
    n, c = x.shape
    src = edge_index[0]
    pad = NW * NPW * K - n * K
    idx = jnp.concatenate([src, jnp.zeros((pad,), jnp.int32)])
    idx = idx.reshape(NW, CHUNKS, CH_ROWS)
    m = _segmax(idx, x)[:n]
    wa = (W[:, :c] - W[:, c:]).T
    wb = W[:, c:].T
    return _matmul(x, m, wa, wb, b.reshape(1, -1))


# speedup vs baseline: 5.0831x; 3.9721x over previous
import functools

import jax
import jax.numpy as jnp
from jax import lax
from jax.experimental import pallas as pl
from jax.experimental.pallas import tpu as pltpu
from jax.experimental.pallas import tpu_sc as plsc

N = 10000
C = 128
K = 32
NW = 32
CH_NODES = 4
CH_ROWS = CH_NODES * K
NPW = 320
CHUNKS = NPW // CH_NODES
LANES = 16


def _tree_max(vals):
    while len(vals) > 1:
        nxt = [jnp.maximum(vals[2 * i], vals[2 * i + 1])
               for i in range(len(vals) // 2)]
        if len(vals) % 2:
            nxt.append(vals[-1])
        vals = nxt
    return vals[0]


def _segmax_body(idx_hbm, x_hbm, out_hbm, idx_v, gbuf, obuf, xs,
                 gsem0, gsem1, osem0, osem1):
    nc = 2
    sid = lax.axis_index("s")
    wid = sid * nc + lax.axis_index("c")
    gsems = (gsem0, gsem1)
    osems = (osem0, osem1)

    @pl.when(sid == 0)
    def _stage():
        pltpu.sync_copy(x_hbm, xs)

    pltpu.sync_copy(idx_hbm.at[wid], idx_v)
    plsc.subcore_barrier()

    for p in range(2):
        pltpu.async_copy(xs.at[idx_v.at[p]], gbuf.at[p], gsems[p])

    def pair(g, carry):
        for p in range(2):
            j = 2 * g + p
            pltpu.make_async_copy(xs.at[idx_v.at[j]], gbuf.at[p],
                                  gsems[p]).wait()

            @pl.when(j >= 2)
            def _owait():
                pltpu.make_async_copy(
                    obuf.at[p],
                    out_hbm.at[pl.ds(wid * NPW, CH_NODES)],
                    osems[p]).wait()

            def cc_body(cc, cc_carry):
                sl = pl.ds(cc * LANES, LANES)
                for nd in range(CH_NODES):
                    acc = None
                    for r0 in range(0, K, 8):
                        grp = _tree_max(
                            [gbuf[p, nd * K + r0 + r, sl] for r in range(8)])
                        acc = grp if acc is None else jnp.maximum(acc, grp)
                    obuf[p, nd, sl] = acc
                return cc_carry

            lax.fori_loop(0, C // LANES, cc_body, 0)

            pltpu.async_copy(
                obuf.at[p],
                out_hbm.at[pl.ds(wid * NPW + j * CH_NODES, CH_NODES)],
                osems[p])

            @pl.when(j + 2 < CHUNKS)
            def _next():
                pltpu.async_copy(xs.at[idx_v.at[j + 2]], gbuf.at[p],
                                 gsems[p])
        return carry

    lax.fori_loop(0, CHUNKS // 2, pair, 0)
    for p in range(2):
        pltpu.make_async_copy(obuf.at[p],
                              out_hbm.at[pl.ds(wid * NPW, CH_NODES)],
                              osems[p]).wait()


_segmax = functools.partial(
    pl.kernel,
    out_type=jax.ShapeDtypeStruct((NW * NPW, C), jnp.float32),
    mesh=plsc.VectorSubcoreMesh(core_axis_name="c", subcore_axis_name="s"),
    scratch_types=[
        pltpu.VMEM((CHUNKS, CH_ROWS), jnp.int32),
        pltpu.VMEM((2, CH_ROWS, C), jnp.float32),
        pltpu.VMEM((2, CH_NODES, C), jnp.float32),
        pltpu.VMEM_SHARED((N, C), jnp.float32),
        pltpu.SemaphoreType.DMA,
        pltpu.SemaphoreType.DMA,
        pltpu.SemaphoreType.DMA,
        pltpu.SemaphoreType.DMA,
    ],
)(_segmax_body)


def _mm_body(x_ref, m_ref, wa_ref, wb_ref, b_ref, o_ref):
    acc = jnp.dot(x_ref[...], wa_ref[...], preferred_element_type=jnp.float32)
    acc = acc + jnp.dot(m_ref[...], wb_ref[...], preferred_element_type=jnp.float32)
    o_ref[...] = jnp.maximum(acc + b_ref[...], 0.0)


def _matmul(x, m, wa, wb, b2d):
    blk = 1000
    grid = (N // blk,)
    return pl.pallas_call(
        _mm_body,
        grid=grid,
        in_specs=[
            pl.BlockSpec((blk, C), lambda i: (i, 0)),
            pl.BlockSpec((blk, C), lambda i: (i, 0)),
            pl.BlockSpec((C, C), lambda i: (0, 0)),
            pl.BlockSpec((C, C), lambda i: (0, 0)),
            pl.BlockSpec((1, C), lambda i: (0, 0)),
        ],
        out_specs=pl.BlockSpec((blk, C), lambda i: (i, 0)),
        out_shape=jax.ShapeDtypeStruct((N, C), jnp.float32),
    )(x, m, wa, wb, b2d)


def kernel(x, edge_index, W, b):
    n, c = x.shape
    src = edge_index[0]
    pad = NW * NPW * K - n * K
    idx = jnp.concatenate([src, jnp.zeros((pad,), jnp.int32)])
    idx = idx.reshape(NW, CHUNKS, CH_ROWS)
    m = _segmax(idx, x)[:n]
    wa = (W[:, :c] - W[:, c:]).T
    wb = W[:, c:].T
    return _matmul(x, m, wa, wb, b.reshape(1, -1))


# speedup vs baseline: 5.2832x; 1.0394x over previous
import functools

import jax
import jax.numpy as jnp
from jax import lax
from jax.experimental import pallas as pl
from jax.experimental.pallas import tpu as pltpu
from jax.experimental.pallas import tpu_sc as plsc

N = 10000
C = 128
K = 32
NW = 32
CH_NODES = 4
CH_ROWS = CH_NODES * K
NPW = 320
CHUNKS = NPW // CH_NODES
LANES = 16
CP = C // 2


def _tree_max(vals):
    while len(vals) > 1:
        nxt = [jnp.maximum(vals[2 * i], vals[2 * i + 1])
               for i in range(len(vals) // 2)]
        if len(vals) % 2:
            nxt.append(vals[-1])
        vals = nxt
    return vals[0]


def _segmax_body(idx_hbm, x_hbm, out_hbm, idx_v, gbuf, obuf, xs, gsem, osem):
    nc = 2
    sid = lax.axis_index("s")
    wid = sid * nc + lax.axis_index("c")

    @pl.when(sid == 0)
    def _stage():
        pltpu.sync_copy(x_hbm, xs)

    pltpu.sync_copy(idx_hbm.at[wid], idx_v)
    plsc.subcore_barrier()

    for p in range(2):
        pltpu.async_copy(xs.at[idx_v.at[p]], gbuf.at[p], gsem.at[p])

    def chunk(j, carry):
        p = lax.rem(j, 2)
        pltpu.make_async_copy(xs.at[idx_v.at[j]], gbuf.at[p],
                              gsem.at[p]).wait()

        @pl.when(j >= 2)
        def _owait():
            pltpu.make_async_copy(
                obuf.at[p],
                out_hbm.at[pl.ds(wid * NPW, CH_NODES)],
                osem.at[p]).wait()

        for nd in range(CH_NODES):
            for cc in range(C // LANES):
                sl = pl.ds(cc * LANES, LANES)
                acc = None
                for r0 in range(0, K, 8):
                    grp = _tree_max(
                        [gbuf[p, nd * K + r0 + r, sl] for r in range(8)])
                    acc = grp if acc is None else jnp.maximum(acc, grp)
                obuf[p, nd, sl] = acc

        pltpu.async_copy(
            obuf.at[p],
            out_hbm.at[pl.ds(wid * NPW + j * CH_NODES, CH_NODES)],
            osem.at[p])

        @pl.when(j + 2 < CHUNKS)
        def _next():
            pltpu.async_copy(xs.at[idx_v.at[j + 2]], gbuf.at[p],
                             gsem.at[p])

        return carry

    lax.fori_loop(0, CHUNKS, chunk, 0)
    for p in range(2):
        pltpu.make_async_copy(obuf.at[p],
                              out_hbm.at[pl.ds(wid * NPW, CH_NODES)],
                              osem.at[p]).wait()


_segmax = functools.partial(
    pl.kernel,
    out_type=jax.ShapeDtypeStruct((NW * NPW, C), jnp.float32),
    mesh=plsc.VectorSubcoreMesh(core_axis_name="c", subcore_axis_name="s"),
    scratch_types=[
        pltpu.VMEM((CHUNKS, CH_ROWS), jnp.int32),
        pltpu.VMEM((2, CH_ROWS, C), jnp.float32),
        pltpu.VMEM((2, CH_NODES, C), jnp.float32),
        pltpu.VMEM_SHARED((N, C), jnp.float32),
        pltpu.SemaphoreType.DMA((2,)),
        pltpu.SemaphoreType.DMA((2,)),
    ],
)(_segmax_body)


def _mm_body(x_ref, m_ref, wa_ref, wb_ref, b_ref, o_ref):
    acc = jnp.dot(x_ref[...], wa_ref[...], preferred_element_type=jnp.float32)
    acc = acc + jnp.dot(m_ref[...], wb_ref[...],
                        preferred_element_type=jnp.float32)
    o_ref[...] = jnp.maximum(acc + b_ref[...], 0.0)


def _matmul(x, m, wa, wb, b2d):
    blk = 1000
    grid = (N // blk,)
    return pl.pallas_call(
        _mm_body,
        grid=grid,
        in_specs=[
            pl.BlockSpec((blk, C), lambda i: (i, 0)),
            pl.BlockSpec((blk, C), lambda i: (i, 0)),
            pl.BlockSpec((C, C), lambda i: (0, 0)),
            pl.BlockSpec((C, C), lambda i: (0, 0)),
            pl.BlockSpec((1, C), lambda i: (0, 0)),
        ],
        out_specs=pl.BlockSpec((blk, C), lambda i: (i, 0)),
        out_shape=jax.ShapeDtypeStruct((N, C), jnp.float32),
    )(x, m, wa, wb, b2d)


def kernel(x, edge_index, W, b):
    n, c = x.shape
    src = edge_index[0]
    pad = NW * NPW * K - n * K
    idx = jnp.concatenate([src, jnp.zeros((pad,), jnp.int32)])
    idx = idx.reshape(NW, CHUNKS, CH_ROWS)
    m = _segmax(idx, x)[:n]
    wa = (W[:, :c] - W[:, c:]).T
    wb = W[:, c:].T
    return _matmul(x, m, wa, wb, b.reshape(1, -1))


# speedup vs baseline: 5.7637x; 1.0910x over previous
import functools

import jax
import jax.numpy as jnp
from jax import lax
from jax.experimental import pallas as pl
from jax.experimental.pallas import tpu as pltpu
from jax.experimental.pallas import tpu_sc as plsc

N = 10000
C = 128
K = 32
NW = 32
CH_NODES = 4
CH_ROWS = CH_NODES * K
NPW = 320
CHUNKS = NPW // CH_NODES
LANES = 16
CP = C // 2


def _tree_max(vals):
    while len(vals) > 1:
        nxt = [jnp.maximum(vals[2 * i], vals[2 * i + 1])
               for i in range(len(vals) // 2)]
        if len(vals) % 2:
            nxt.append(vals[-1])
        vals = nxt
    return vals[0]


NBUF = 2


def _segmax_body(idx_hbm, x_hbm, out_hbm, idx_v, gbuf, obuf, xs, gsem, osem):
    nc = 2
    sid = lax.axis_index("s")
    wid = sid * nc + lax.axis_index("c")

    @pl.when(sid == 0)
    def _stage():
        pltpu.sync_copy(x_hbm, xs)

    pltpu.sync_copy(idx_hbm.at[wid], idx_v)
    plsc.subcore_barrier()

    for p in range(2):
        pltpu.async_copy(xs.at[idx_v.at[p]], gbuf.at[p], gsem.at[p])

    def chunk(j, carry):
        p = lax.rem(j, NBUF)
        pltpu.make_async_copy(xs.at[idx_v.at[j]], gbuf.at[p],
                              gsem.at[p]).wait()

        @pl.when(j >= NBUF)
        def _owait():
            pltpu.make_async_copy(
                obuf.at[p],
                out_hbm.at[pl.ds(wid * NPW, CH_NODES)],
                osem.at[p]).wait()

        for nd in range(CH_NODES):
            for cc in range(CP // LANES):
                sl = pl.ds(cc * LANES, LANES)
                acc_h = None
                acc_l = None
                for r0 in range(0, K, 8):
                    ws = [gbuf[p, nd * K + r0 + r, sl] for r in range(8)]
                    gh = _tree_max(ws)
                    gl = _tree_max([lax.shift_left(w, 16) for w in ws])
                    acc_h = gh if acc_h is None else jnp.maximum(acc_h, gh)
                    acc_l = gl if acc_l is None else jnp.maximum(acc_l, gl)
                obuf[p, nd, sl] = jnp.bitwise_or(
                    jnp.bitwise_and(acc_h, jnp.int32(-65536)),
                    lax.shift_right_logical(acc_l, 16))

        pltpu.async_copy(
            obuf.at[p],
            out_hbm.at[pl.ds(wid * NPW + j * CH_NODES, CH_NODES)],
            osem.at[p])

        @pl.when(j + NBUF < CHUNKS)
        def _next():
            pltpu.async_copy(xs.at[idx_v.at[j + NBUF]], gbuf.at[p],
                             gsem.at[p])

        return carry

    lax.fori_loop(0, CHUNKS, chunk, 0)
    for p in range(NBUF):
        pltpu.make_async_copy(obuf.at[p],
                              out_hbm.at[pl.ds(wid * NPW, CH_NODES)],
                              osem.at[p]).wait()


_segmax = functools.partial(
    pl.kernel,
    out_type=jax.ShapeDtypeStruct((NW * NPW, CP), jnp.int32),
    mesh=plsc.VectorSubcoreMesh(core_axis_name="c", subcore_axis_name="s"),
    scratch_types=[
        pltpu.VMEM((CHUNKS, CH_ROWS), jnp.int32),
        pltpu.VMEM((NBUF, CH_ROWS, CP), jnp.int32),
        pltpu.VMEM((NBUF, CH_NODES, CP), jnp.int32),
        pltpu.VMEM_SHARED((N, CP), jnp.int32),
        pltpu.SemaphoreType.DMA((NBUF,)),
        pltpu.SemaphoreType.DMA((NBUF,)),
    ],
)(_segmax_body)


def _decode_codes(c16):
    neg = jnp.bitwise_and(c16, 0x8000) != 0
    bits = jnp.where(
        neg,
        jnp.bitwise_and(jnp.bitwise_not(jnp.bitwise_xor(c16, 0x8000)),
                        0xFFFF),
        c16)
    return lax.bitcast_convert_type(lax.shift_left(bits, 16), jnp.float32)


def _mm_body(x_ref, mc_ref, wa_ref, wbp_ref, b_ref, o_ref):
    w = mc_ref[...]
    m_lo = _decode_codes(jnp.bitwise_and(w, 0xFFFF))
    m_hi = _decode_codes(lax.shift_right_logical(w, 16))
    acc = jnp.dot(x_ref[...], wa_ref[...], preferred_element_type=jnp.float32)
    acc = acc + jnp.dot(jnp.concatenate([m_lo, m_hi], axis=1), wbp_ref[...],
                        preferred_element_type=jnp.float32)
    o_ref[...] = jnp.maximum(acc + b_ref[...], 0.0)


def _matmul(x, m_codes, wa, wbp, b2d):
    blk = 1000
    grid = (N // blk,)
    return pl.pallas_call(
        _mm_body,
        grid=grid,
        in_specs=[
            pl.BlockSpec((blk, C), lambda i: (i, 0)),
            pl.BlockSpec((blk, CP), lambda i: (i, 0)),
            pl.BlockSpec((C, C), lambda i: (0, 0)),
            pl.BlockSpec((C, C), lambda i: (0, 0)),
            pl.BlockSpec((1, C), lambda i: (0, 0)),
        ],
        out_specs=pl.BlockSpec((blk, C), lambda i: (i, 0)),
        out_shape=jax.ShapeDtypeStruct((N, C), jnp.float32),
    )(x, m_codes, wa, wbp, b2d)


def kernel(x, edge_index, W, b):
    n, c = x.shape
    src = edge_index[0]
    pad = NW * NPW * K - n * K
    idx = jnp.concatenate([src, jnp.zeros((pad,), jnp.int32)])
    idx = idx.reshape(NW, CHUNKS, CH_ROWS)
    bits = lax.bitcast_convert_type(x.astype(jnp.bfloat16), jnp.uint16)
    neg = jnp.bitwise_and(bits, jnp.uint16(0x8000)) != 0
    code = jnp.where(
        neg,
        jnp.bitwise_xor(jnp.bitwise_not(bits), jnp.uint16(0x8000)),
        bits)
    x_codes = lax.bitcast_convert_type(code.reshape(n, CP, 2), jnp.int32)
    m_codes = _segmax(idx, x_codes)
    wa = (W[:, :c] - W[:, c:]).T
    wb = W[:, c:].T
    wbp = jnp.concatenate([wb[0::2], wb[1::2]], axis=0)
    return _matmul(x, m_codes, wa, wbp, b.reshape(1, -1))


# speedup vs baseline: 5.7694x; 1.0010x over previous
import functools

import jax
import jax.numpy as jnp
from jax import lax
from jax.experimental import pallas as pl
from jax.experimental.pallas import tpu as pltpu
from jax.experimental.pallas import tpu_sc as plsc

N = 10000
C = 128
K = 32
NW = 32
CH_NODES = 4
CH_ROWS = CH_NODES * K
NPW = 320
CHUNKS = NPW // CH_NODES
LANES = 16
CP = C // 2


def _tree_max(vals):
    while len(vals) > 1:
        nxt = [jnp.maximum(vals[2 * i], vals[2 * i + 1])
               for i in range(len(vals) // 2)]
        if len(vals) % 2:
            nxt.append(vals[-1])
        vals = nxt
    return vals[0]


NBUF = 2


def _segmax_body(idx_hbm, x_hbm, out_hbm, idx_v, gbuf, obuf, xs, gsem, osem):
    nc = 2
    sid = lax.axis_index("s")
    wid = sid * nc + lax.axis_index("c")

    @pl.when(sid == 0)
    def _stage():
        pltpu.sync_copy(x_hbm, xs)

    pltpu.sync_copy(idx_hbm.at[wid], idx_v)
    plsc.subcore_barrier()

    for p in range(2):
        pltpu.async_copy(xs.at[idx_v.at[p]], gbuf.at[p], gsem.at[p])

    def chunk(j, carry):
        p = lax.rem(j, NBUF)
        pltpu.make_async_copy(xs.at[idx_v.at[j]], gbuf.at[p],
                              gsem.at[p]).wait()

        @pl.when(j >= NBUF)
        def _owait():
            pltpu.make_async_copy(
                obuf.at[p],
                out_hbm.at[pl.ds(wid * NPW, CH_NODES)],
                osem.at[p]).wait()

        for nd in range(CH_NODES):
            for cc in range(CP // LANES):
                sl = pl.ds(cc * LANES, LANES)
                acc_h = None
                acc_l = None
                for r0 in range(0, K, 8):
                    ws = [gbuf[p, nd * K + r0 + r, sl] for r in range(8)]
                    gh = _tree_max(ws)
                    gl = _tree_max([lax.shift_left(w, 16) for w in ws])
                    acc_h = gh if acc_h is None else jnp.maximum(acc_h, gh)
                    acc_l = gl if acc_l is None else jnp.maximum(acc_l, gl)
                obuf[p, nd, sl] = jnp.bitwise_or(
                    jnp.bitwise_and(acc_h, jnp.int32(-65536)),
                    lax.shift_right_logical(acc_l, 16))

        pltpu.async_copy(
            obuf.at[p],
            out_hbm.at[pl.ds(wid * NPW + j * CH_NODES, CH_NODES)],
            osem.at[p])

        @pl.when(j + NBUF < CHUNKS)
        def _next():
            pltpu.async_copy(xs.at[idx_v.at[j + NBUF]], gbuf.at[p],
                             gsem.at[p])

        return carry

    lax.fori_loop(0, CHUNKS, chunk, 0)
    for p in range(NBUF):
        pltpu.make_async_copy(obuf.at[p],
                              out_hbm.at[pl.ds(wid * NPW, CH_NODES)],
                              osem.at[p]).wait()


_segmax = functools.partial(
    pl.kernel,
    out_type=jax.ShapeDtypeStruct((NW * NPW, CP), jnp.int32),
    mesh=plsc.VectorSubcoreMesh(core_axis_name="c", subcore_axis_name="s"),
    scratch_types=[
        pltpu.VMEM((CHUNKS, CH_ROWS), jnp.int32),
        pltpu.VMEM((NBUF, CH_ROWS, CP), jnp.int32),
        pltpu.VMEM((NBUF, CH_NODES, CP), jnp.int32),
        pltpu.VMEM_SHARED((N, CP), jnp.int32),
        pltpu.SemaphoreType.DMA((NBUF,)),
        pltpu.SemaphoreType.DMA((NBUF,)),
    ],
)(_segmax_body)


def _decode_codes(c16):
    neg = jnp.bitwise_and(c16, 0x8000) != 0
    bits = jnp.where(
        neg,
        jnp.bitwise_and(jnp.bitwise_not(jnp.bitwise_xor(c16, 0x8000)),
                        0xFFFF),
        c16)
    return lax.bitcast_convert_type(lax.shift_left(bits, 16), jnp.float32)


_MM_BLK = 1000


def _mm1_body(x_ref, wa_ref, b_ref, o_ref):
    o_ref[...] = jnp.dot(x_ref[...], wa_ref[...],
                         preferred_element_type=jnp.float32) + b_ref[...]


def _mm1(x, wa, b2d):
    return pl.pallas_call(
        _mm1_body,
        grid=(N // _MM_BLK,),
        in_specs=[
            pl.BlockSpec((_MM_BLK, C), lambda i: (i, 0)),
            pl.BlockSpec((C, C), lambda i: (0, 0)),
            pl.BlockSpec((1, C), lambda i: (0, 0)),
        ],
        out_specs=pl.BlockSpec((_MM_BLK, C), lambda i: (i, 0)),
        out_shape=jax.ShapeDtypeStruct((N, C), jnp.float32),
    )(x, wa, b2d)


def _mm2_body(p_ref, mc_ref, wbp_ref, o_ref):
    w = mc_ref[...]
    m_lo = _decode_codes(jnp.bitwise_and(w, 0xFFFF))
    m_hi = _decode_codes(lax.shift_right_logical(w, 16))
    acc = p_ref[...] + jnp.dot(jnp.concatenate([m_lo, m_hi], axis=1),
                               wbp_ref[...],
                               preferred_element_type=jnp.float32)
    o_ref[...] = jnp.maximum(acc, 0.0)


def _mm2(p, m_codes, wbp):
    return pl.pallas_call(
        _mm2_body,
        grid=(N // _MM_BLK,),
        in_specs=[
            pl.BlockSpec((_MM_BLK, C), lambda i: (i, 0)),
            pl.BlockSpec((_MM_BLK, CP), lambda i: (i, 0)),
            pl.BlockSpec((C, C), lambda i: (0, 0)),
        ],
        out_specs=pl.BlockSpec((_MM_BLK, C), lambda i: (i, 0)),
        out_shape=jax.ShapeDtypeStruct((N, C), jnp.float32),
    )(p, m_codes, wbp)


def kernel(x, edge_index, W, b):
    n, c = x.shape
    src = edge_index[0]
    pad = NW * NPW * K - n * K
    idx = jnp.concatenate([src, jnp.zeros((pad,), jnp.int32)])
    idx = idx.reshape(NW, CHUNKS, CH_ROWS)
    bits = lax.bitcast_convert_type(x.astype(jnp.bfloat16), jnp.uint16)
    neg = jnp.bitwise_and(bits, jnp.uint16(0x8000)) != 0
    code = jnp.where(
        neg,
        jnp.bitwise_xor(jnp.bitwise_not(bits), jnp.uint16(0x8000)),
        bits)
    x_codes = lax.bitcast_convert_type(code.reshape(n, CP, 2), jnp.int32)
    m_codes = _segmax(idx, x_codes)
    wa = (W[:, :c] - W[:, c:]).T
    wb = W[:, c:].T
    wbp = jnp.concatenate([wb[0::2], wb[1::2]], axis=0)
    p = _mm1(x, wa, b.reshape(1, -1))
    return _mm2(p, m_codes, wbp)


# speedup vs baseline: 6.8371x; 1.1851x over previous
import functools

import jax
import jax.numpy as jnp
from jax import lax
from jax.experimental import pallas as pl
from jax.experimental.pallas import tpu as pltpu
from jax.experimental.pallas import tpu_sc as plsc

N = 10000
C = 128
K = 32
NW = 32
CH_NODES = 4
CH_ROWS = CH_NODES * K
NPW = 320
CHUNKS = NPW // CH_NODES
LANES = 16
CP = C // 2


def _tree_max(vals):
    while len(vals) > 1:
        nxt = [jnp.maximum(vals[2 * i], vals[2 * i + 1])
               for i in range(len(vals) // 2)]
        if len(vals) % 2:
            nxt.append(vals[-1])
        vals = nxt
    return vals[0]


NBUF = 2


def _segmax_body(idx_hbm, x_hbm, out_hbm, idx_v, gbuf, obuf, xs, gsem, osem):
    nc = 2
    sid = lax.axis_index("s")
    wid = sid * nc + lax.axis_index("c")

    @pl.when(sid == 0)
    def _stage():
        pltpu.sync_copy(x_hbm, xs)

    pltpu.sync_copy(idx_hbm.at[wid], idx_v)
    plsc.subcore_barrier()

    for p in range(2):
        pltpu.async_copy(xs.at[idx_v.at[p]], gbuf.at[p], gsem.at[p])

    def chunk(j, carry):
        p = lax.rem(j, NBUF)
        pltpu.make_async_copy(xs.at[idx_v.at[j]], gbuf.at[p],
                              gsem.at[p]).wait()

        @pl.when(j >= NBUF)
        def _owait():
            pltpu.make_async_copy(
                obuf.at[p],
                out_hbm.at[pl.ds(wid * NPW, CH_NODES)],
                osem.at[p]).wait()

        for nd in range(CH_NODES):
            for cc in range(CP // LANES):
                sl = pl.ds(cc * LANES, LANES)
                acc_h = None
                acc_l = None
                for r0 in range(0, K, 8):
                    ws = [gbuf[p, nd * K + r0 + r, sl] for r in range(8)]
                    gh = _tree_max(ws)
                    gl = _tree_max([lax.shift_left(w, 16) for w in ws])
                    acc_h = gh if acc_h is None else jnp.maximum(acc_h, gh)
                    acc_l = gl if acc_l is None else jnp.maximum(acc_l, gl)
                obuf[p, nd, sl] = jnp.bitwise_or(
                    jnp.bitwise_and(acc_h, jnp.int32(-65536)),
                    lax.shift_right_logical(acc_l, 16))

        pltpu.async_copy(
            obuf.at[p],
            out_hbm.at[pl.ds(wid * NPW + j * CH_NODES, CH_NODES)],
            osem.at[p])

        @pl.when(j + NBUF < CHUNKS)
        def _next():
            pltpu.async_copy(xs.at[idx_v.at[j + NBUF]], gbuf.at[p],
                             gsem.at[p])

        return carry

    lax.fori_loop(0, CHUNKS, chunk, 0)
    for p in range(NBUF):
        pltpu.make_async_copy(obuf.at[p],
                              out_hbm.at[pl.ds(wid * NPW, CH_NODES)],
                              osem.at[p]).wait()


_segmax = functools.partial(
    pl.kernel,
    out_type=jax.ShapeDtypeStruct((NW * NPW, CP), jnp.int32),
    mesh=plsc.VectorSubcoreMesh(core_axis_name="c", subcore_axis_name="s"),
    scratch_types=[
        pltpu.VMEM((CHUNKS, CH_ROWS), jnp.int32),
        pltpu.VMEM((NBUF, CH_ROWS, CP), jnp.int32),
        pltpu.VMEM((NBUF, CH_NODES, CP), jnp.int32),
        pltpu.VMEM_SHARED((N, CP), jnp.int32),
        pltpu.SemaphoreType.DMA((NBUF,)),
        pltpu.SemaphoreType.DMA((NBUF,)),
    ],
)(_segmax_body)


def _decode_codes(c16):
    neg = jnp.bitwise_and(c16, 0x8000) != 0
    bits = jnp.where(
        neg,
        jnp.bitwise_and(jnp.bitwise_not(jnp.bitwise_xor(c16, 0x8000)),
                        0xFFFF),
        c16)
    return lax.bitcast_convert_type(lax.shift_left(bits, 16), jnp.float32)


_MM_BLK = 1000


def _enc_body(x_ref, o_ref):
    bits = lax.bitcast_convert_type(x_ref[...].astype(jnp.bfloat16),
                                    jnp.uint16).astype(jnp.int32)
    neg = jnp.bitwise_and(bits, 0x8000) != 0
    code = jnp.where(
        neg,
        jnp.bitwise_and(jnp.bitwise_xor(jnp.bitwise_not(bits), 0x8000),
                        0xFFFF),
        bits)
    o_ref[...] = jnp.bitwise_or(code[:, :CP],
                                lax.shift_left(code[:, CP:], 16))


def _encode(x):
    return pl.pallas_call(
        _enc_body,
        grid=(N // _MM_BLK,),
        in_specs=[pl.BlockSpec((_MM_BLK, C), lambda i: (i, 0))],
        out_specs=pl.BlockSpec((_MM_BLK, CP), lambda i: (i, 0)),
        out_shape=jax.ShapeDtypeStruct((N, CP), jnp.int32),
    )(x)


def _mm1_body(x_ref, wa_ref, b_ref, o_ref):
    o_ref[...] = jnp.dot(x_ref[...], wa_ref[...],
                         preferred_element_type=jnp.float32) + b_ref[...]


def _mm1(x, wa, b2d):
    return pl.pallas_call(
        _mm1_body,
        grid=(N // _MM_BLK,),
        in_specs=[
            pl.BlockSpec((_MM_BLK, C), lambda i: (i, 0)),
            pl.BlockSpec((C, C), lambda i: (0, 0)),
            pl.BlockSpec((1, C), lambda i: (0, 0)),
        ],
        out_specs=pl.BlockSpec((_MM_BLK, C), lambda i: (i, 0)),
        out_shape=jax.ShapeDtypeStruct((N, C), jnp.float32),
    )(x, wa, b2d)


def _mm2_body(p_ref, mc_ref, wbp_ref, o_ref):
    w = mc_ref[...]
    m_lo = _decode_codes(jnp.bitwise_and(w, 0xFFFF))
    m_hi = _decode_codes(lax.shift_right_logical(w, 16))
    acc = p_ref[...] + jnp.dot(jnp.concatenate([m_lo, m_hi], axis=1),
                               wbp_ref[...],
                               preferred_element_type=jnp.float32)
    o_ref[...] = jnp.maximum(acc, 0.0)


def _mm2(p, m_codes, wbp):
    return pl.pallas_call(
        _mm2_body,
        grid=(N // _MM_BLK,),
        in_specs=[
            pl.BlockSpec((_MM_BLK, C), lambda i: (i, 0)),
            pl.BlockSpec((_MM_BLK, CP), lambda i: (i, 0)),
            pl.BlockSpec((C, C), lambda i: (0, 0)),
        ],
        out_specs=pl.BlockSpec((_MM_BLK, C), lambda i: (i, 0)),
        out_shape=jax.ShapeDtypeStruct((N, C), jnp.float32),
    )(p, m_codes, wbp)


def kernel(x, edge_index, W, b):
    n, c = x.shape
    src = edge_index[0]
    pad = NW * NPW * K - n * K
    idx = jnp.concatenate([src, jnp.zeros((pad,), jnp.int32)])
    idx = idx.reshape(NW, CHUNKS, CH_ROWS)
    x_codes = _encode(x)
    m_codes = _segmax(idx, x_codes)
    wa = (W[:, :c] - W[:, c:]).T
    wb = W[:, c:].T
    p = _mm1(x, wa, b.reshape(1, -1))
    return _mm2(p, m_codes, wb)


# speedup vs baseline: 7.8877x; 1.1537x over previous
import functools

import jax
import jax.numpy as jnp
from jax import lax
from jax.experimental import pallas as pl
from jax.experimental.pallas import tpu as pltpu
from jax.experimental.pallas import tpu_sc as plsc

N = 10000
C = 128
K = 32
NW = 32
CH_NODES = 4
CH_ROWS = CH_NODES * K
NPW = 320
CHUNKS = NPW // CH_NODES
LANES = 16
CP = C // 2


def _tree_max(vals):
    while len(vals) > 1:
        nxt = [jnp.maximum(vals[2 * i], vals[2 * i + 1])
               for i in range(len(vals) // 2)]
        if len(vals) % 2:
            nxt.append(vals[-1])
        vals = nxt
    return vals[0]


NBUF = 2
TAIL_W = N // NPW
TAIL_NODES = N - TAIL_W * NPW
TAIL_CHUNKS = TAIL_NODES // CH_NODES
TAIL_E = TAIL_NODES * K


def _segmax_body(edge_hbm, x_hbm, out_hbm, idx_v, gbuf, obuf, xs, gsem, osem):
    nc = 2
    sid = lax.axis_index("s")
    wid = sid * nc + lax.axis_index("c")

    @pl.when(sid == 0)
    def _stage():
        pltpu.sync_copy(x_hbm, xs)

    @pl.when(wid < TAIL_W)
    def _load_idx():
        pltpu.sync_copy(edge_hbm.at[0, pl.ds(wid * NPW * K, NPW * K)], idx_v)

    @pl.when(wid == TAIL_W)
    def _load_idx_tail():
        pltpu.sync_copy(edge_hbm.at[0, pl.ds(TAIL_W * NPW * K, TAIL_E)],
                        idx_v.at[pl.ds(0, TAIL_E)])

    nchunks = jnp.where(wid == TAIL_W, TAIL_CHUNKS, CHUNKS)
    plsc.subcore_barrier()

    for p in range(2):
        pltpu.async_copy(xs.at[idx_v.at[pl.ds(p * CH_ROWS, CH_ROWS)]],
                         gbuf.at[p], gsem.at[p])

    def chunk(j, carry):
        p = lax.rem(j, NBUF)
        pltpu.make_async_copy(xs.at[idx_v.at[pl.ds(j * CH_ROWS, CH_ROWS)]],
                              gbuf.at[p], gsem.at[p]).wait()

        @pl.when(j >= NBUF)
        def _owait():
            pltpu.make_async_copy(
                obuf.at[p],
                out_hbm.at[pl.ds(wid * NPW, CH_NODES)],
                osem.at[p]).wait()

        for nd in range(CH_NODES):
            for cc in range(CP // LANES):
                sl = pl.ds(cc * LANES, LANES)
                acc_h = None
                acc_l = None
                for r0 in range(0, K, 8):
                    ws = [gbuf[p, nd * K + r0 + r, sl] for r in range(8)]
                    gh = _tree_max(ws)
                    gl = _tree_max([lax.shift_left(w, 16) for w in ws])
                    acc_h = gh if acc_h is None else jnp.maximum(acc_h, gh)
                    acc_l = gl if acc_l is None else jnp.maximum(acc_l, gl)
                obuf[p, nd, sl] = jnp.bitwise_or(
                    jnp.bitwise_and(acc_h, jnp.int32(-65536)),
                    lax.shift_right_logical(acc_l, 16))

        pltpu.async_copy(
            obuf.at[p],
            out_hbm.at[pl.ds(wid * NPW + j * CH_NODES, CH_NODES)],
            osem.at[p])

        @pl.when(j + NBUF < nchunks)
        def _next():
            pltpu.async_copy(
                xs.at[idx_v.at[pl.ds((j + NBUF) * CH_ROWS, CH_ROWS)]],
                gbuf.at[p], gsem.at[p])

        return carry

    lax.fori_loop(0, nchunks, chunk, 0)
    for p in range(NBUF):
        pltpu.make_async_copy(obuf.at[p],
                              out_hbm.at[pl.ds(wid * NPW, CH_NODES)],
                              osem.at[p]).wait()


_segmax = functools.partial(
    pl.kernel,
    out_type=jax.ShapeDtypeStruct((NW * NPW, CP), jnp.int32),
    mesh=plsc.VectorSubcoreMesh(core_axis_name="c", subcore_axis_name="s"),
    scratch_types=[
        pltpu.VMEM((NPW * K,), jnp.int32),
        pltpu.VMEM((NBUF, CH_ROWS, CP), jnp.int32),
        pltpu.VMEM((NBUF, CH_NODES, CP), jnp.int32),
        pltpu.VMEM_SHARED((N, CP), jnp.int32),
        pltpu.SemaphoreType.DMA((NBUF,)),
        pltpu.SemaphoreType.DMA((NBUF,)),
    ],
)(_segmax_body)


def _decode_codes(c16):
    neg = jnp.bitwise_and(c16, 0x8000) != 0
    bits = jnp.where(
        neg,
        jnp.bitwise_and(jnp.bitwise_not(jnp.bitwise_xor(c16, 0x8000)),
                        0xFFFF),
        c16)
    return lax.bitcast_convert_type(lax.shift_left(bits, 16), jnp.float32)


_MM_BLK = 1000


def _enc_body(x_ref, o_ref):
    bits = lax.bitcast_convert_type(x_ref[...].astype(jnp.bfloat16),
                                    jnp.uint16).astype(jnp.int32)
    neg = jnp.bitwise_and(bits, 0x8000) != 0
    code = jnp.where(
        neg,
        jnp.bitwise_and(jnp.bitwise_xor(jnp.bitwise_not(bits), 0x8000),
                        0xFFFF),
        bits)
    o_ref[...] = jnp.bitwise_or(code[:, :CP],
                                lax.shift_left(code[:, CP:], 16))


def _encode(x):
    return pl.pallas_call(
        _enc_body,
        grid=(N // _MM_BLK,),
        in_specs=[pl.BlockSpec((_MM_BLK, C), lambda i: (i, 0))],
        out_specs=pl.BlockSpec((_MM_BLK, CP), lambda i: (i, 0)),
        out_shape=jax.ShapeDtypeStruct((N, CP), jnp.int32),
    )(x)


def _mm1_body(x_ref, wa_ref, b_ref, o_ref):
    o_ref[...] = jnp.dot(x_ref[...], wa_ref[...],
                         preferred_element_type=jnp.float32) + b_ref[...]


def _mm1(x, wa, b2d):
    return pl.pallas_call(
        _mm1_body,
        grid=(N // _MM_BLK,),
        in_specs=[
            pl.BlockSpec((_MM_BLK, C), lambda i: (i, 0)),
            pl.BlockSpec((C, C), lambda i: (0, 0)),
            pl.BlockSpec((1, C), lambda i: (0, 0)),
        ],
        out_specs=pl.BlockSpec((_MM_BLK, C), lambda i: (i, 0)),
        out_shape=jax.ShapeDtypeStruct((N, C), jnp.float32),
    )(x, wa, b2d)


def _mm2_body(p_ref, mc_ref, wbp_ref, o_ref):
    w = mc_ref[...]
    m_lo = _decode_codes(jnp.bitwise_and(w, 0xFFFF))
    m_hi = _decode_codes(lax.shift_right_logical(w, 16))
    acc = p_ref[...] + jnp.dot(jnp.concatenate([m_lo, m_hi], axis=1),
                               wbp_ref[...],
                               preferred_element_type=jnp.float32)
    o_ref[...] = jnp.maximum(acc, 0.0)


def _mm2(p, m_codes, wbp):
    return pl.pallas_call(
        _mm2_body,
        grid=(N // _MM_BLK,),
        in_specs=[
            pl.BlockSpec((_MM_BLK, C), lambda i: (i, 0)),
            pl.BlockSpec((_MM_BLK, CP), lambda i: (i, 0)),
            pl.BlockSpec((C, C), lambda i: (0, 0)),
        ],
        out_specs=pl.BlockSpec((_MM_BLK, C), lambda i: (i, 0)),
        out_shape=jax.ShapeDtypeStruct((N, C), jnp.float32),
    )(p, m_codes, wbp)


def kernel(x, edge_index, W, b):
    n, c = x.shape
    x_codes = _encode(x)
    m_codes = _segmax(edge_index, x_codes)
    wa = (W[:, :c] - W[:, c:]).T
    wb = W[:, c:].T
    p = _mm1(x, wa, b.reshape(1, -1))
    return _mm2(p, m_codes, wb)
